# baseline (speedup 1.0000x reference)
import math

import jax
import jax.numpy as jnp
from jax import lax
from jax.experimental import pallas as pl
from jax.experimental.pallas import tpu as pltpu
from jax.experimental.pallas import tpu_sc as plsc

D_M = 32
SCALE = math.sqrt(32.0)
V = 1000001
V_FULL = (V // 128) * 128
N_BLK = V // 128
R_ROWS = 250008

NC = 2
NS = 16
NW = NC * NS

BPW = N_BLK // NW
N_REM = N_BLK - BPW * NW


def _iota16():
    return lax.iota(jnp.int32, 16)


def _conv_body(tabT, tail, rowtab, src_v, dst_v, tail_v, si0, si1, so0, so1):
    sem_i = (si0, si1)
    sem_o = (so0, so1)
    wid = lax.axis_index("s") * NC + lax.axis_index("c")
    b0 = wid * BPW

    def fire_load(vb, p):
        pltpu.async_copy(tabT.at[:, pl.ds(vb * 128, 128)], src_v.at[p],
                         sem_i[p])

    def wait_load(p):
        pltpu.make_async_copy(tabT.at[:, pl.ds(0, 128)], src_v.at[p],
                              sem_i[p]).wait()

    def wait_store(p):
        pltpu.make_async_copy(dst_v.at[p], rowtab.at[pl.ds(0, 32)],
                              sem_o[p]).wait()

    def transpose(p):
        @plsc.parallel_loop(0, 32, step=1, unroll=2)
        def _(r):
            for h in range(4):
                col = jnp.broadcast_to(4 * r + h, (16,)).astype(jnp.int32)
                for par in range(2):
                    rows16 = 16 * par + _iota16()
                    vals = plsc.load_gather(src_v.at[p], [rows16, col])
                    dst_v[p, r, pl.ds(32 * h + 16 * par, 16)] = vals * SCALE

    def fire_store(vb, p):
        pltpu.async_copy(dst_v.at[p], rowtab.at[pl.ds(vb * 32, 32)], sem_o[p])

    def step(vb, p, fire_next2, wait_prev_store):
        wait_load(p)
        if wait_prev_store:
            wait_store(p)
        transpose(p)
        if fire_next2:
            fire_load(vb + 2, p)
        fire_store(vb, p)

    fire_load(b0, 0)
    fire_load(b0 + 1, 1)
    step(b0, 0, True, False)
    step(b0 + 1, 1, True, False)

    def body(j, c):
        step(b0 + 2 * j, 0, True, True)
        step(b0 + 2 * j + 1, 1, True, True)
        return c

    lax.fori_loop(1, BPW // 2 - 1, body, 0)
    step(b0 + BPW - 2, 0, False, True)
    step(b0 + BPW - 1, 1, False, True)
    wait_store(0)
    wait_store(1)

    @pl.when(wid < N_REM)
    def _():
        vb = NW * BPW + wid
        fire_load(vb, 0)
        wait_load(0)
        transpose(0)
        fire_store(vb, 0)
        wait_store(0)

    @pl.when(wid == N_REM)
    def _():
        pltpu.sync_copy(tail, tail_v)
        pltpu.sync_copy(tail_v, rowtab.at[pl.ds((V_FULL // 4), 24)])


def _gather_body(idxT, rowtab, out3, idx_v, q_v, g_v, o_v,
                 sg0, sg1, sg2, sg3, so0, so1):
    sem_g = (sg0, sg1, sg2, sg3)
    sem_o = (so0, so1)
    wid = lax.axis_index("s") * NC + lax.axis_index("c")
    t0 = wid * 100
    u0 = t0 * 8

    def load_idx(t):
        tr = t // 128
        tc = lax.rem(t, 128)
        pltpu.sync_copy(idxT.at[pl.ds(tr * 8, 8), pl.ds(tc * 128, 128)],
                        idx_v.at[lax.rem(t, 2)])

    def prep_and_fire(u, gp):
        t = u // 8
        s2 = lax.rem(u, 8)

        @pl.when(lax.rem(u, 8) == 0)
        def _():
            load_idx(t)

        q = lax.rem(t, 2)
        for cb in range(8):
            iv = idx_v[q, s2, pl.ds(cb * 16, 16)]
            q_v[gp, 0, pl.ds(cb * 16, 16)] = lax.shift_right_logical(iv, 2)
            q_v[gp, 1, pl.ds(cb * 16, 16)] = lax.shift_left(
                jnp.bitwise_and(iv, 3), 5)
        pltpu.async_copy(rowtab.at[q_v.at[gp, 0]], g_v.at[gp], sem_g[gp])

    def drain_gather(gp):
        pltpu.make_async_copy(rowtab.at[pl.ds(0, 128)], g_v.at[gp],
                              sem_g[gp]).wait()

    def wait_store(op):
        pltpu.make_async_copy(o_v.at[op], out3.at[0, :, pl.ds(0, 128)],
                              sem_o[op]).wait()

    def select_store(u, gp, op):
        @plsc.parallel_loop(0, 8, step=1, unroll=4)
        def _(cb):
            rows16 = 16 * cb + _iota16()
            off16 = q_v[gp, 1, pl.ds(cb * 16, 16)]
            for f in range(32):
                vals = plsc.load_gather(g_v.at[gp], [rows16, off16 + f])
                o_v[op, f, pl.ds(cb * 16, 16)] = vals

        t = u // 8
        s2 = lax.rem(u, 8)
        tr = t // 128
        tc = lax.rem(t, 128)
        pltpu.async_copy(o_v.at[op],
                         out3.at[tr * 8 + s2, :, pl.ds(tc * 128, 128)],
                         sem_o[op])

    def step(u, r, fire, wait_o):
        if fire:
            prep_and_fire(u + 2, (r + 2) % 4)
        drain_gather(r % 4)
        if wait_o:
            wait_store(r % 2)
        select_store(u, r % 4, r % 2)

    prep_and_fire(u0, 0)
    prep_and_fire(u0 + 1, 1)
    for r in range(4):
        step(u0 + r, r, True, r >= 2)

    def body(j, c):
        for r in range(4):
            step(u0 + 4 * j + r, r, True, True)
        return c

    lax.fori_loop(1, 199, body, 0)
    for r in range(4):
        step(u0 + 796 + r, r, r < 2, True)
    wait_store(0)
    wait_store(1)


def kernel(input, table):
    B0, S = input.shape
    idxT = input.T.astype(jnp.int32)
    tabT = table.T
    tail = jnp.pad((table[V_FULL:] * SCALE).reshape(-1),
                   (0, 24 * 128 - (V - V_FULL) * D_M)).reshape(24, 128)

    mesh = plsc.VectorSubcoreMesh(core_axis_name="c", subcore_axis_name="s")
    tiled = pltpu.CompilerParams(use_tc_tiling_on_sc=True,
                                 needs_layout_passes=False)

    conv = pl.kernel(
        _conv_body,
        out_type=jax.ShapeDtypeStruct((R_ROWS, 128), jnp.float32),
        mesh=mesh,
        scratch_types=[
            pltpu.VMEM((2, 32, 128), jnp.float32),
            pltpu.VMEM((2, 32, 128), jnp.float32),
            pltpu.VMEM((24, 128), jnp.float32),
        ] + [pltpu.SemaphoreType.DMA] * 4,
        compiler_params=tiled,
    )
    rowtab = conv(tabT, tail)

    gat = pl.kernel(
        _gather_body,
        out_type=jax.ShapeDtypeStruct((S, D_M, B0), jnp.float32),
        mesh=mesh,
        scratch_types=[
            pltpu.VMEM((2, 8, 128), jnp.int32),
            pltpu.VMEM((4, 2, 128), jnp.int32),
            pltpu.VMEM((4, 128, 128), jnp.float32),
            pltpu.VMEM((2, D_M, 128), jnp.float32),
        ] + [pltpu.SemaphoreType.DMA] * 6,
        compiler_params=tiled,
    )
    out3 = gat(idxT, rowtab)
    return jnp.transpose(out3, (2, 0, 1))


# speedup vs baseline: 1.7516x; 1.7516x over previous
import math

import jax
import jax.numpy as jnp
from jax import lax
from jax.experimental import pallas as pl
from jax.experimental.pallas import tpu as pltpu
from jax.experimental.pallas import tpu_sc as plsc

D_M = 32
SCALE = math.sqrt(32.0)
V = 1000001
V_FULL = (V // 128) * 128
N_BLK = V // 128
R_ROWS = 250008

NC = 2
NS = 16
NW = NC * NS

BPW = N_BLK // NW
N_REM = N_BLK - BPW * NW


def _iota16():
    return lax.iota(jnp.int32, 16)


def _conv_body(tabT, tail, rowtab, src_v, dst_v, tail_v, si0, si1, so0, so1):
    sem_i = (si0, si1)
    sem_o = (so0, so1)
    wid = lax.axis_index("s") * NC + lax.axis_index("c")
    b0 = wid * BPW

    def fire_load(vb, p):
        pltpu.async_copy(tabT.at[:, pl.ds(vb * 128, 128)], src_v.at[p],
                         sem_i[p])

    def wait_load(p):
        pltpu.make_async_copy(tabT.at[:, pl.ds(0, 128)], src_v.at[p],
                              sem_i[p]).wait()

    def wait_store(p):
        pltpu.make_async_copy(dst_v.at[p], rowtab.at[pl.ds(0, 32)],
                              sem_o[p]).wait()

    def transpose(p):
        @plsc.parallel_loop(0, 8, step=1, unroll=2)
        def _(grp):
            v16 = 16 * grp + _iota16()
            rowv = lax.shift_right_logical(v16, 2)
            colb = lax.shift_left(jnp.bitwise_and(v16, 3), 5)
            for d in range(32):
                fvec = jnp.bitwise_and(d + _iota16(), 31)
                vals = plsc.load_gather(src_v.at[p], [fvec, v16])
                plsc.store_scatter(dst_v.at[p], [rowv, colb + fvec],
                                   vals * SCALE)

    def fire_store(vb, p):
        pltpu.async_copy(dst_v.at[p], rowtab.at[pl.ds(vb * 32, 32)], sem_o[p])

    def step(vb, p, fire_next2, wait_prev_store):
        wait_load(p)
        if wait_prev_store:
            wait_store(p)
        transpose(p)
        if fire_next2:
            fire_load(vb + 2, p)
        fire_store(vb, p)

    fire_load(b0, 0)
    fire_load(b0 + 1, 1)
    step(b0, 0, True, False)
    step(b0 + 1, 1, True, False)

    def body(j, c):
        step(b0 + 2 * j, 0, True, True)
        step(b0 + 2 * j + 1, 1, True, True)
        return c

    lax.fori_loop(1, BPW // 2 - 1, body, 0)
    step(b0 + BPW - 2, 0, False, True)
    step(b0 + BPW - 1, 1, False, True)
    wait_store(0)
    wait_store(1)

    @pl.when(wid < N_REM)
    def _():
        vb = NW * BPW + wid
        fire_load(vb, 0)
        wait_load(0)
        transpose(0)
        fire_store(vb, 0)
        wait_store(0)

    @pl.when(wid == N_REM)
    def _():
        pltpu.sync_copy(tail, tail_v)
        pltpu.sync_copy(tail_v, rowtab.at[pl.ds((V_FULL // 4), 24)])


def _gather_body(idxT, rowtab, out3, idx_v, q_v, g_v, o_v,
                 sg0, sg1, sg2, sg3, so0, so1):
    sem_g = (sg0, sg1, sg2, sg3)
    sem_o = (so0, so1)
    wid = lax.axis_index("s") * NC + lax.axis_index("c")
    t0 = wid * 100
    u0 = t0 * 8

    def load_idx(t):
        tr = t // 128
        tc = lax.rem(t, 128)
        pltpu.sync_copy(idxT.at[pl.ds(tr * 8, 8), pl.ds(tc * 128, 128)],
                        idx_v.at[lax.rem(t, 2)])

    def prep_and_fire(u, gp):
        t = u // 8
        s2 = lax.rem(u, 8)

        @pl.when(lax.rem(u, 8) == 0)
        def _():
            load_idx(t)

        q = lax.rem(t, 2)
        for cb in range(8):
            iv = idx_v[q, s2, pl.ds(cb * 16, 16)]
            q_v[gp, 0, pl.ds(cb * 16, 16)] = lax.shift_right_logical(iv, 2)
            q_v[gp, 1, pl.ds(cb * 16, 16)] = lax.shift_left(
                jnp.bitwise_and(iv, 3), 5)
        pltpu.async_copy(rowtab.at[q_v.at[gp, 0]], g_v.at[gp], sem_g[gp])

    def drain_gather(gp):
        pltpu.make_async_copy(rowtab.at[pl.ds(0, 128)], g_v.at[gp],
                              sem_g[gp]).wait()

    def wait_store(op):
        pltpu.make_async_copy(o_v.at[op], out3.at[0, :, pl.ds(0, 128)],
                              sem_o[op]).wait()

    def select_store(u, gp, op):
        @plsc.parallel_loop(0, 8, step=1, unroll=2)
        def _(cb):
            bvec = 16 * cb + _iota16()
            off16 = q_v[gp, 1, pl.ds(cb * 16, 16)]
            for d in range(32):
                fvec = jnp.bitwise_and(d + _iota16(), 31)
                vals = plsc.load_gather(g_v.at[gp], [bvec, off16 + fvec])
                plsc.store_scatter(o_v.at[op], [fvec, bvec], vals)

        t = u // 8
        s2 = lax.rem(u, 8)
        tr = t // 128
        tc = lax.rem(t, 128)
        pltpu.async_copy(o_v.at[op],
                         out3.at[tr * 8 + s2, :, pl.ds(tc * 128, 128)],
                         sem_o[op])

    def step(u, r, fire, wait_o):
        if fire:
            prep_and_fire(u + 2, (r + 2) % 4)
        drain_gather(r % 4)
        if wait_o:
            wait_store(r % 2)
        select_store(u, r % 4, r % 2)

    prep_and_fire(u0, 0)
    prep_and_fire(u0 + 1, 1)
    for r in range(4):
        step(u0 + r, r, True, r >= 2)

    def body(j, c):
        for r in range(4):
            step(u0 + 4 * j + r, r, True, True)
        return c

    lax.fori_loop(1, 199, body, 0)
    for r in range(4):
        step(u0 + 796 + r, r, r < 2, True)
    wait_store(0)
    wait_store(1)


def kernel(input, table):
    B0, S = input.shape
    idxT = input.T.astype(jnp.int32)
    tabT = table.T
    tail = jnp.pad((table[V_FULL:] * SCALE).reshape(-1),
                   (0, 24 * 128 - (V - V_FULL) * D_M)).reshape(24, 128)

    mesh = plsc.VectorSubcoreMesh(core_axis_name="c", subcore_axis_name="s")
    tiled = pltpu.CompilerParams(use_tc_tiling_on_sc=True,
                                 needs_layout_passes=False)

    conv = pl.kernel(
        _conv_body,
        out_type=jax.ShapeDtypeStruct((R_ROWS, 128), jnp.float32),
        mesh=mesh,
        scratch_types=[
            pltpu.VMEM((2, 32, 128), jnp.float32),
            pltpu.VMEM((2, 32, 128), jnp.float32),
            pltpu.VMEM((24, 128), jnp.float32),
        ] + [pltpu.SemaphoreType.DMA] * 4,
        compiler_params=tiled,
    )
    rowtab = conv(tabT, tail)

    gat = pl.kernel(
        _gather_body,
        out_type=jax.ShapeDtypeStruct((S, D_M, B0), jnp.float32),
        mesh=mesh,
        scratch_types=[
            pltpu.VMEM((2, 8, 128), jnp.int32),
            pltpu.VMEM((4, 2, 128), jnp.int32),
            pltpu.VMEM((4, 128, 128), jnp.float32),
            pltpu.VMEM((2, D_M, 128), jnp.float32),
        ] + [pltpu.SemaphoreType.DMA] * 6,
        compiler_params=tiled,
    )
    out3 = gat(idxT, rowtab)
    return jnp.transpose(out3, (2, 0, 1))
